# baseline (speedup 1.0000x reference)
import jax
import jax.numpy as jnp
from jax import lax
from jax.experimental import pallas as pl
from jax.experimental.pallas import tpu as pltpu
from jax.experimental.pallas import tpu_sc as plsc

_NC = 2
_NS = 16
_NW = _NC * _NS
_CHUNK = 128


def _make_gather(B, D, dtype):
    b_per_w = B // _NW
    n_chunks = b_per_w // _CHUNK
    mesh = plsc.VectorSubcoreMesh(core_axis_name="c", subcore_axis_name="s")

    def body(jids, cids, jtab, ctab, jout, cout, jidx, cidx, jrows, crows, sem):
        wid = lax.axis_index("s") * _NC + lax.axis_index("c")
        base = wid * b_per_w
        pltpu.sync_copy(jids.at[pl.ds(base, b_per_w)], jidx)
        pltpu.sync_copy(cids.at[pl.ds(base, b_per_w)], cidx)
        copies = []
        for j in range(n_chunks):
            s = pl.ds(j * _CHUNK, _CHUNK)
            copies.append(pltpu.async_copy(jtab.at[jidx.at[s]], jrows.at[s], sem))
            copies.append(pltpu.async_copy(ctab.at[cidx.at[s]], crows.at[s], sem))
        for c in copies:
            c.wait()
        pltpu.sync_copy(jrows, jout.at[pl.ds(base, b_per_w)])
        pltpu.sync_copy(crows, cout.at[pl.ds(base, b_per_w)])

    return pl.kernel(
        body,
        out_type=(
            jax.ShapeDtypeStruct((B, D), dtype),
            jax.ShapeDtypeStruct((B, D), dtype),
        ),
        mesh=mesh,
        scratch_types=[
            pltpu.VMEM((b_per_w,), jnp.int32),
            pltpu.VMEM((b_per_w,), jnp.int32),
            pltpu.VMEM((b_per_w, D), dtype),
            pltpu.VMEM((b_per_w, D), dtype),
            pltpu.SemaphoreType.DMA,
        ],
        compiler_params=pltpu.CompilerParams(use_tc_tiling_on_sc=False),
    )


def _mlp_body(xj_ref, xc_ref, w1_ref, b1_ref, w2_ref, b2_ref, w3_ref, b3_ref,
              out_ref):
    d = xj_ref.shape[1]
    w1 = w1_ref[...]
    h = xj_ref[...] @ w1[:d, :] + xc_ref[...] @ w1[d:, :] + b1_ref[...]
    h = jnp.maximum(h, 0.0)
    h = jnp.maximum(h @ w2_ref[...] + b2_ref[...], 0.0)
    s = jax.nn.sigmoid(h @ w3_ref[...] + b3_ref[...])
    out_ref[...] = s[:, 0]


def kernel(job_ids, candidate_ids, job_table, cand_table, W1, b1, W2, b2, W3,
           b3):
    B = job_ids.shape[0]
    D = job_table.shape[1]
    H1 = W1.shape[1]
    H2 = W2.shape[1]

    gather = _make_gather(B, D, job_table.dtype)
    job_emb, cand_emb = gather(job_ids, candidate_ids, job_table, cand_table)

    blk = 2048
    grid = (B // blk,)
    score = pl.pallas_call(
        _mlp_body,
        grid=grid,
        in_specs=[
            pl.BlockSpec((blk, D), lambda i: (i, 0)),
            pl.BlockSpec((blk, D), lambda i: (i, 0)),
            pl.BlockSpec((2 * D, H1), lambda i: (0, 0)),
            pl.BlockSpec((1, H1), lambda i: (0, 0)),
            pl.BlockSpec((H1, H2), lambda i: (0, 0)),
            pl.BlockSpec((1, H2), lambda i: (0, 0)),
            pl.BlockSpec((H2, 1), lambda i: (0, 0)),
            pl.BlockSpec((1, 1), lambda i: (0, 0)),
        ],
        out_specs=pl.BlockSpec((blk,), lambda i: (i,)),
        out_shape=jax.ShapeDtypeStruct((B,), jnp.float32),
    )(job_emb, cand_emb, W1, b1.reshape(1, H1), W2, b2.reshape(1, H2), W3,
      b3.reshape(1, 1))
    return score


# speedup vs baseline: 1.5834x; 1.5834x over previous
import jax
import jax.numpy as jnp
from jax import lax
from jax.experimental import pallas as pl
from jax.experimental.pallas import tpu as pltpu
from jax.experimental.pallas import tpu_sc as plsc

_NC = 2
_NS = 16
_NW = _NC * _NS
_LAG = 16


def _make_gather(B, D, dtype):
    b_per_w = B // _NW
    mesh = plsc.VectorSubcoreMesh(core_axis_name="c", subcore_axis_name="s")

    def body(jids, cids, jtab, ctab, jout, cout, vidx, rows, sem):
        wid = lax.axis_index("s") * _NC + lax.axis_index("c")
        base = wid * b_per_w
        n_groups = b_per_w // 16

        for ids, tab, out in ((jids, jtab, jout), (cids, ctab, cout)):
            pltpu.sync_copy(ids.at[pl.ds(base, b_per_w)], vidx)

            def group_body(g, _):
                v = vidx[pl.ds(g * 16, 16)]
                for l in range(16):
                    rid = lax.squeeze(lax.slice(v, (l,), (l + 1,)), (0,))
                    pltpu.make_async_copy(
                        tab.at[pl.ds(rid, 1)],
                        rows.at[pl.ds(g * 16 + l, 1)], sem).start()

                @pl.when(g >= 1)
                def _():
                    for l in range(16):
                        pltpu.make_async_copy(
                            tab.at[pl.ds(0, 1)],
                            rows.at[pl.ds((g - 1) * 16 + l, 1)], sem).wait()

                return 0

            lax.fori_loop(0, n_groups, group_body, 0)
            for l in range(16):
                pltpu.make_async_copy(
                    tab.at[pl.ds(0, 1)],
                    rows.at[pl.ds((n_groups - 1) * 16 + l, 1)], sem).wait()
            pltpu.sync_copy(rows, out.at[pl.ds(base, b_per_w)])

    return pl.kernel(
        body,
        out_type=(
            jax.ShapeDtypeStruct((B, D), dtype),
            jax.ShapeDtypeStruct((B, D), dtype),
        ),
        mesh=mesh,
        scratch_types=[
            pltpu.VMEM((b_per_w,), jnp.int32),
            pltpu.VMEM((b_per_w, D), dtype),
            pltpu.SemaphoreType.DMA,
        ],
    )


def _mlp_body(xj_ref, xc_ref, w1_ref, b1_ref, w2_ref, b2_ref, w3_ref, b3_ref,
              out_ref):
    d = xj_ref.shape[1]
    w1 = w1_ref[...]
    h = xj_ref[...] @ w1[:d, :] + xc_ref[...] @ w1[d:, :] + b1_ref[...]
    h = jnp.maximum(h, 0.0)
    h = jnp.maximum(h @ w2_ref[...] + b2_ref[...], 0.0)
    s = jax.nn.sigmoid(h @ w3_ref[...] + b3_ref[...])
    out_ref[...] = s[:, 0]


def kernel(job_ids, candidate_ids, job_table, cand_table, W1, b1, W2, b2, W3,
           b3):
    B = job_ids.shape[0]
    D = job_table.shape[1]
    H1 = W1.shape[1]
    H2 = W2.shape[1]

    gather = _make_gather(B, D, job_table.dtype)
    job_emb, cand_emb = gather(job_ids, candidate_ids, job_table, cand_table)

    blk = 2048
    grid = (B // blk,)
    score = pl.pallas_call(
        _mlp_body,
        grid=grid,
        in_specs=[
            pl.BlockSpec((blk, D), lambda i: (i, 0)),
            pl.BlockSpec((blk, D), lambda i: (i, 0)),
            pl.BlockSpec((2 * D, H1), lambda i: (0, 0)),
            pl.BlockSpec((1, H1), lambda i: (0, 0)),
            pl.BlockSpec((H1, H2), lambda i: (0, 0)),
            pl.BlockSpec((1, H2), lambda i: (0, 0)),
            pl.BlockSpec((H2, 1), lambda i: (0, 0)),
            pl.BlockSpec((1, 1), lambda i: (0, 0)),
        ],
        out_specs=pl.BlockSpec((blk,), lambda i: (i,)),
        out_shape=jax.ShapeDtypeStruct((B,), jnp.float32),
    )(job_emb, cand_emb, W1, b1.reshape(1, H1), W2, b2.reshape(1, H2), W3,
      b3.reshape(1, 1))
    return score


# speedup vs baseline: 1.6164x; 1.0208x over previous
import jax
import jax.numpy as jnp
from jax import lax
from jax.experimental import pallas as pl
from jax.experimental.pallas import tpu as pltpu
from jax.experimental.pallas import tpu_sc as plsc

_NC = 2
_NS = 16
_NW = _NC * _NS


def _make_gather(B, D, dtype):
    b_per_w = B // _NW
    n_groups = b_per_w // 16
    mesh = plsc.VectorSubcoreMesh(core_axis_name="c", subcore_axis_name="s")

    def body(jids, cids, jtab, ctab, jout, cout, vjdx, vcdx, jrows, crows,
             sem):
        wid = lax.axis_index("s") * _NC + lax.axis_index("c")
        base = wid * b_per_w
        half = b_per_w // 2
        hg = n_groups // 2

        pltpu.sync_copy(jids.at[pl.ds(base, b_per_w)], vjdx)
        pltpu.sync_copy(cids.at[pl.ds(base, b_per_w)], vcdx)

        for p in range(2):
            def group_body(g, _):
                vj = vjdx[pl.ds(p * half + g * 16, 16)]
                vc = vcdx[pl.ds(p * half + g * 16, 16)]
                for l in range(16):
                    jid = lax.squeeze(lax.slice(vj, (l,), (l + 1,)), (0,))
                    cid = lax.squeeze(lax.slice(vc, (l,), (l + 1,)), (0,))
                    pltpu.make_async_copy(
                        jtab.at[pl.ds(jid, 1)],
                        jrows.at[pl.ds(g * 16 + l, 1)], sem).start()
                    pltpu.make_async_copy(
                        ctab.at[pl.ds(cid, 1)],
                        crows.at[pl.ds(g * 16 + l, 1)], sem).start()

                @pl.when(g >= 1)
                def _():
                    for l in range(16):
                        pltpu.make_async_copy(
                            jtab.at[pl.ds(0, 1)],
                            jrows.at[pl.ds((g - 1) * 16 + l, 1)], sem).wait()
                        pltpu.make_async_copy(
                            ctab.at[pl.ds(0, 1)],
                            crows.at[pl.ds((g - 1) * 16 + l, 1)], sem).wait()

                return 0

            lax.fori_loop(0, hg, group_body, 0)
            for l in range(16):
                pltpu.make_async_copy(
                    jtab.at[pl.ds(0, 1)],
                    jrows.at[pl.ds((hg - 1) * 16 + l, 1)], sem).wait()
                pltpu.make_async_copy(
                    ctab.at[pl.ds(0, 1)],
                    crows.at[pl.ds((hg - 1) * 16 + l, 1)], sem).wait()
            pltpu.sync_copy(jrows, jout.at[pl.ds(base + p * half, half)])
            pltpu.sync_copy(crows, cout.at[pl.ds(base + p * half, half)])

    return pl.kernel(
        body,
        out_type=(
            jax.ShapeDtypeStruct((B, D), dtype),
            jax.ShapeDtypeStruct((B, D), dtype),
        ),
        mesh=mesh,
        scratch_types=[
            pltpu.VMEM((b_per_w,), jnp.int32),
            pltpu.VMEM((b_per_w,), jnp.int32),
            pltpu.VMEM((b_per_w // 2, D), dtype),
            pltpu.VMEM((b_per_w // 2, D), dtype),
            pltpu.SemaphoreType.DMA,
        ],
    )


def _mlp_body(xj_ref, xc_ref, w1_ref, b1_ref, w2_ref, b2_ref, w3_ref, b3_ref,
              out_ref):
    d = xj_ref.shape[1]
    w1 = w1_ref[...]
    h = xj_ref[...] @ w1[:d, :] + xc_ref[...] @ w1[d:, :] + b1_ref[...]
    h = jnp.maximum(h, 0.0)
    h = jnp.maximum(h @ w2_ref[...] + b2_ref[...], 0.0)
    s = jax.nn.sigmoid(h @ w3_ref[...] + b3_ref[...])
    out_ref[...] = s[:, 0]


def kernel(job_ids, candidate_ids, job_table, cand_table, W1, b1, W2, b2, W3,
           b3):
    B = job_ids.shape[0]
    D = job_table.shape[1]
    H1 = W1.shape[1]
    H2 = W2.shape[1]

    gather = _make_gather(B, D, job_table.dtype)
    job_emb, cand_emb = gather(job_ids, candidate_ids, job_table, cand_table)

    blk = 4096
    grid = (B // blk,)
    score = pl.pallas_call(
        _mlp_body,
        grid=grid,
        in_specs=[
            pl.BlockSpec((blk, D), lambda i: (i, 0)),
            pl.BlockSpec((blk, D), lambda i: (i, 0)),
            pl.BlockSpec((2 * D, H1), lambda i: (0, 0)),
            pl.BlockSpec((1, H1), lambda i: (0, 0)),
            pl.BlockSpec((H1, H2), lambda i: (0, 0)),
            pl.BlockSpec((1, H2), lambda i: (0, 0)),
            pl.BlockSpec((H2, 1), lambda i: (0, 0)),
            pl.BlockSpec((1, 1), lambda i: (0, 0)),
        ],
        out_specs=pl.BlockSpec((blk,), lambda i: (i,)),
        out_shape=jax.ShapeDtypeStruct((B,), jnp.float32),
    )(job_emb, cand_emb, W1, b1.reshape(1, H1), W2, b2.reshape(1, H2), W3,
      b3.reshape(1, 1))
    return score


# speedup vs baseline: 1.7662x; 1.0927x over previous
import jax
import jax.numpy as jnp
from jax import lax
from jax.experimental import pallas as pl
from jax.experimental.pallas import tpu as pltpu
from jax.experimental.pallas import tpu_sc as plsc

_NC = 2
_NS = 16
_NW = _NC * _NS


def _make_gather(B, D, dtype):
    b_per_w = B // _NW
    n_groups = b_per_w // 16
    mesh = plsc.VectorSubcoreMesh(core_axis_name="c", subcore_axis_name="s")

    def body(jids, cids, jtab3, ctab3, jout, cout, vjdx, vcdx, jblk, cblk,
             jrows, crows, sem):
        wid = lax.axis_index("s") * _NC + lax.axis_index("c")
        base = wid * b_per_w
        half = b_per_w // 4
        hg = n_groups // 4
        lanes = lax.iota(jnp.int32, 16)
        zeros = lanes * 0

        pltpu.sync_copy(jids.at[pl.ds(base, b_per_w)], vjdx)
        pltpu.sync_copy(cids.at[pl.ds(base, b_per_w)], vcdx)

        def _extract(v, blocks, rows, g):
            for l in range(16):
                s = g * 16 + l
                rid = lax.squeeze(lax.slice(v, (l,), (l + 1,)), (0,))
                slot = (g % 2) * 16 + l
                r = rid & 15
                i0 = zeros + s
                for c in range(D // 32):
                    ab = blocks[slot, r, pl.ds(c * 32, 32)]
                    a, b = plsc.unpack(ab, format=plsc.PackFormat.INTERLEAVED)
                    plsc.store_scatter(rows, [i0, c * 32 + 2 * lanes], a)
                    plsc.store_scatter(rows, [i0, c * 32 + 2 * lanes + 1], b)

        for p in range(4):
            def group_body(g, _):
                vj = vjdx[pl.ds(p * half + g * 16, 16)]
                vc = vcdx[pl.ds(p * half + g * 16, 16)]
                for l in range(16):
                    jid = lax.squeeze(lax.slice(vj, (l,), (l + 1,)), (0,))
                    cid = lax.squeeze(lax.slice(vc, (l,), (l + 1,)), (0,))
                    slot = (g % 2) * 16 + l
                    pltpu.make_async_copy(
                        jtab3.at[jid >> 4], jblk.at[slot], sem).start()
                    pltpu.make_async_copy(
                        ctab3.at[cid >> 4], cblk.at[slot], sem).start()

                @pl.when(g >= 1)
                def _():
                    for l in range(16):
                        slot = ((g - 1) % 2) * 16 + l
                        pltpu.make_async_copy(
                            jtab3.at[0], jblk.at[slot], sem).wait()
                        pltpu.make_async_copy(
                            ctab3.at[0], cblk.at[slot], sem).wait()
                    vjp = vjdx[pl.ds(p * half + (g - 1) * 16, 16)]
                    vcp = vcdx[pl.ds(p * half + (g - 1) * 16, 16)]
                    _extract(vjp, jblk, jrows, g - 1)
                    _extract(vcp, cblk, crows, g - 1)

                return 0

            lax.fori_loop(0, hg, group_body, 0)
            for l in range(16):
                slot = ((hg - 1) % 2) * 16 + l
                pltpu.make_async_copy(jtab3.at[0], jblk.at[slot], sem).wait()
                pltpu.make_async_copy(ctab3.at[0], cblk.at[slot], sem).wait()
            vjp = vjdx[pl.ds(p * half + (hg - 1) * 16, 16)]
            vcp = vcdx[pl.ds(p * half + (hg - 1) * 16, 16)]
            _extract(vjp, jblk, jrows, hg - 1)
            _extract(vcp, cblk, crows, hg - 1)

            pltpu.sync_copy(jrows, jout.at[pl.ds(base + p * half, half)])
            pltpu.sync_copy(crows, cout.at[pl.ds(base + p * half, half)])

    return pl.kernel(
        body,
        out_type=(
            jax.ShapeDtypeStruct((B, D), jnp.float32),
            jax.ShapeDtypeStruct((B, D), jnp.float32),
        ),
        mesh=mesh,
        scratch_types=[
            pltpu.VMEM((b_per_w,), jnp.int32),
            pltpu.VMEM((b_per_w,), jnp.int32),
            pltpu.VMEM((32, 16, D), dtype),
            pltpu.VMEM((32, 16, D), dtype),
            pltpu.VMEM((b_per_w // 4, D), jnp.float32),
            pltpu.VMEM((b_per_w // 4, D), jnp.float32),
            pltpu.SemaphoreType.DMA,
        ],
        compiler_params=pltpu.CompilerParams(needs_layout_passes=False),
    )


def _mlp_body(xj_ref, xc_ref, w1_ref, b1_ref, w2_ref, b2_ref, w3_ref, b3_ref,
              out_ref):
    d = xj_ref.shape[1]
    w1 = w1_ref[...]
    h = xj_ref[...] @ w1[:d, :] + xc_ref[...] @ w1[d:, :] + b1_ref[...]
    h = jnp.maximum(h, 0.0)
    h = jnp.maximum(h @ w2_ref[...] + b2_ref[...], 0.0)
    s = jax.nn.sigmoid(h @ w3_ref[...] + b3_ref[...])
    out_ref[...] = s[:, 0]


def kernel(job_ids, candidate_ids, job_table, cand_table, W1, b1, W2, b2, W3,
           b3):
    B = job_ids.shape[0]
    NJ, D = job_table.shape
    NCAND = cand_table.shape[0]
    H1 = W1.shape[1]
    H2 = W2.shape[1]

    jtab3 = job_table.astype(jnp.bfloat16).reshape(NJ // 16, 16, D)
    ctab3 = cand_table.astype(jnp.bfloat16).reshape(NCAND // 16, 16, D)

    gather = _make_gather(B, D, jnp.bfloat16)
    job_emb, cand_emb = gather(job_ids, candidate_ids, jtab3, ctab3)

    blk = 4096
    grid = (B // blk,)
    score = pl.pallas_call(
        _mlp_body,
        grid=grid,
        in_specs=[
            pl.BlockSpec((blk, D), lambda i: (i, 0)),
            pl.BlockSpec((blk, D), lambda i: (i, 0)),
            pl.BlockSpec((2 * D, H1), lambda i: (0, 0)),
            pl.BlockSpec((1, H1), lambda i: (0, 0)),
            pl.BlockSpec((H1, H2), lambda i: (0, 0)),
            pl.BlockSpec((1, H2), lambda i: (0, 0)),
            pl.BlockSpec((H2, 1), lambda i: (0, 0)),
            pl.BlockSpec((1, 1), lambda i: (0, 0)),
        ],
        out_specs=pl.BlockSpec((blk,), lambda i: (i,)),
        out_shape=jax.ShapeDtypeStruct((B,), jnp.float32),
    )(job_emb, cand_emb, W1, b1.reshape(1, H1), W2, b2.reshape(1, H2), W3,
      b3.reshape(1, 1))
    return score


# speedup vs baseline: 1.7696x; 1.0019x over previous
import jax
import jax.numpy as jnp
from jax import lax
from jax.experimental import pallas as pl
from jax.experimental.pallas import tpu as pltpu
from jax.experimental.pallas import tpu_sc as plsc

_NC = 2
_NS = 16
_NW = _NC * _NS


def _make_gather(B, D, dtype):
    b_per_w = B // _NW
    n_groups = b_per_w // 16
    mesh = plsc.VectorSubcoreMesh(core_axis_name="c", subcore_axis_name="s")

    def body(jids, cids, jtab3, ctab3, jout, cout, vjdx, vcdx, jblk, cblk,
             jrows, crows, sem):
        wid = lax.axis_index("s") * _NC + lax.axis_index("c")
        base = wid * b_per_w
        half = b_per_w // 4
        hg = n_groups // 4
        lanes = lax.iota(jnp.int32, 16)
        zeros = lanes * 0

        pltpu.sync_copy(jids.at[pl.ds(base, b_per_w)], vjdx)
        pltpu.sync_copy(cids.at[pl.ds(base, b_per_w)], vcdx)

        def _extract(v, blocks, rows, g):
            for l in range(16):
                s = g * 16 + l
                rid = lax.squeeze(lax.slice(v, (l,), (l + 1,)), (0,))
                slot = (g % 2) * 16 + l
                r = rid & 15
                i0 = zeros + s
                for c in range(D // 32):
                    ab = blocks[slot, r, pl.ds(c * 32, 32)]
                    a, b = plsc.unpack(ab, format=plsc.PackFormat.INTERLEAVED)
                    plsc.store_scatter(rows, [i0, c * 32 + 2 * lanes], a)
                    plsc.store_scatter(rows, [i0, c * 32 + 2 * lanes + 1], b)

        for p in range(4):
            def group_body(g, _):
                vj = vjdx[pl.ds(p * half + g * 16, 16)]
                vc = vcdx[pl.ds(p * half + g * 16, 16)]
                for l in range(16):
                    jid = lax.squeeze(lax.slice(vj, (l,), (l + 1,)), (0,))
                    cid = lax.squeeze(lax.slice(vc, (l,), (l + 1,)), (0,))
                    slot = (g % 2) * 16 + l
                    pltpu.make_async_copy(
                        jtab3.at[jid >> 4], jblk.at[slot], sem).start()
                    pltpu.make_async_copy(
                        ctab3.at[cid >> 4], cblk.at[slot], sem).start()

                @pl.when(g >= 1)
                def _():
                    for l in range(16):
                        slot = ((g - 1) % 2) * 16 + l
                        pltpu.make_async_copy(
                            jtab3.at[0], jblk.at[slot], sem).wait()
                        pltpu.make_async_copy(
                            ctab3.at[0], cblk.at[slot], sem).wait()
                    vjp = vjdx[pl.ds(p * half + (g - 1) * 16, 16)]
                    vcp = vcdx[pl.ds(p * half + (g - 1) * 16, 16)]
                    _extract(vjp, jblk, jrows, g - 1)
                    _extract(vcp, cblk, crows, g - 1)

                return 0

            lax.fori_loop(0, hg, group_body, 0)
            for l in range(16):
                slot = ((hg - 1) % 2) * 16 + l
                pltpu.make_async_copy(jtab3.at[0], jblk.at[slot], sem).wait()
                pltpu.make_async_copy(ctab3.at[0], cblk.at[slot], sem).wait()
            vjp = vjdx[pl.ds(p * half + (hg - 1) * 16, 16)]
            vcp = vcdx[pl.ds(p * half + (hg - 1) * 16, 16)]
            _extract(vjp, jblk, jrows, hg - 1)
            _extract(vcp, cblk, crows, hg - 1)

            pltpu.sync_copy(jrows, jout.at[pl.ds(base + p * half, half)])
            pltpu.sync_copy(crows, cout.at[pl.ds(base + p * half, half)])

    return pl.kernel(
        body,
        out_type=(
            jax.ShapeDtypeStruct((B, D), jnp.float32),
            jax.ShapeDtypeStruct((B, D), jnp.float32),
        ),
        mesh=mesh,
        scratch_types=[
            pltpu.VMEM((b_per_w,), jnp.int32),
            pltpu.VMEM((b_per_w,), jnp.int32),
            pltpu.VMEM((32, 16, D), dtype),
            pltpu.VMEM((32, 16, D), dtype),
            pltpu.VMEM((b_per_w // 4, D), jnp.float32),
            pltpu.VMEM((b_per_w // 4, D), jnp.float32),
            pltpu.SemaphoreType.DMA,
        ],
        compiler_params=pltpu.CompilerParams(needs_layout_passes=False),
    )


def _mlp_body(xj_ref, xc_ref, w1_ref, b1_ref, w2_ref, b2_ref, w3_ref, b3_ref,
              out_ref):
    d = xj_ref.shape[1]
    w1 = w1_ref[...]
    h = xj_ref[...] @ w1[:d, :] + xc_ref[...] @ w1[d:, :] + b1_ref[...]
    h = jnp.maximum(h, 0.0)
    h = jnp.maximum(h @ w2_ref[...] + b2_ref[...], 0.0)
    s = jax.nn.sigmoid(h @ w3_ref[...] + b3_ref[...])
    out_ref[...] = s[:, 0]


def kernel(job_ids, candidate_ids, job_table, cand_table, W1, b1, W2, b2, W3,
           b3):
    B = job_ids.shape[0]
    NJ, D = job_table.shape
    NCAND = cand_table.shape[0]
    H1 = W1.shape[1]
    H2 = W2.shape[1]

    jtab3 = job_table.astype(jnp.bfloat16).reshape(NJ // 16, 16, D)
    ctab3 = cand_table.astype(jnp.bfloat16).reshape(NCAND // 16, 16, D)

    gather = _make_gather(B, D, jnp.bfloat16)
    job_emb, cand_emb = gather(job_ids, candidate_ids, jtab3, ctab3)

    blk = 8192
    grid = (B // blk,)
    score = pl.pallas_call(
        _mlp_body,
        grid=grid,
        in_specs=[
            pl.BlockSpec((blk, D), lambda i: (i, 0)),
            pl.BlockSpec((blk, D), lambda i: (i, 0)),
            pl.BlockSpec((2 * D, H1), lambda i: (0, 0)),
            pl.BlockSpec((1, H1), lambda i: (0, 0)),
            pl.BlockSpec((H1, H2), lambda i: (0, 0)),
            pl.BlockSpec((1, H2), lambda i: (0, 0)),
            pl.BlockSpec((H2, 1), lambda i: (0, 0)),
            pl.BlockSpec((1, 1), lambda i: (0, 0)),
        ],
        out_specs=pl.BlockSpec((blk,), lambda i: (i,)),
        out_shape=jax.ShapeDtypeStruct((B,), jnp.float32),
    )(job_emb, cand_emb, W1, b1.reshape(1, H1), W2, b2.reshape(1, H2), W3,
      b3.reshape(1, 1))
    return score


# speedup vs baseline: 1.7699x; 1.0002x over previous
import jax
import jax.numpy as jnp
from jax import lax
from jax.experimental import pallas as pl
from jax.experimental.pallas import tpu as pltpu
from jax.experimental.pallas import tpu_sc as plsc

_NC = 2
_NS = 16
_NW = _NC * _NS


def _make_gather(B, D, dtype):
    b_per_w = B // _NW
    n_groups = b_per_w // 16
    mesh = plsc.VectorSubcoreMesh(core_axis_name="c", subcore_axis_name="s")

    def body(jids, cids, jtab3, ctab3, jout, cout, vjdx, vcdx, jblk, cblk,
             jrows, crows, sem):
        wid = lax.axis_index("s") * _NC + lax.axis_index("c")
        base = wid * b_per_w
        half = b_per_w // 4
        hg = n_groups // 4
        lanes = lax.iota(jnp.int32, 16)
        zeros = lanes * 0

        pltpu.sync_copy(jids.at[pl.ds(base, b_per_w)], vjdx)
        pltpu.sync_copy(cids.at[pl.ds(base, b_per_w)], vcdx)

        def _extract(v, blocks, rows, g):
            for l in range(16):
                s = g * 16 + l
                rid = lax.squeeze(lax.slice(v, (l,), (l + 1,)), (0,))
                slot = (g % 2) * 16 + l
                r = rid & 15
                i0 = zeros + s
                for c in range(D // 32):
                    ab = blocks[slot, r, pl.ds(c * 32, 32)]
                    a, b = plsc.unpack(ab, format=plsc.PackFormat.INTERLEAVED)
                    plsc.store_scatter(rows, [i0, c * 32 + 2 * lanes], a)
                    plsc.store_scatter(rows, [i0, c * 32 + 2 * lanes + 1], b)

        for p in range(4):
            def group_body(g, _):
                vj = vjdx[pl.ds(p * half + g * 16, 16)]
                vc = vcdx[pl.ds(p * half + g * 16, 16)]
                for l in range(16):
                    jid = lax.squeeze(lax.slice(vj, (l,), (l + 1,)), (0,))
                    cid = lax.squeeze(lax.slice(vc, (l,), (l + 1,)), (0,))
                    slot = (g % 2) * 16 + l
                    pltpu.make_async_copy(
                        jtab3.at[jid >> 4], jblk.at[slot], sem).start()
                    pltpu.make_async_copy(
                        ctab3.at[cid >> 4], cblk.at[slot], sem).start()

                @pl.when(g >= 1)
                def _():
                    for l in range(16):
                        slot = ((g - 1) % 2) * 16 + l
                        pltpu.make_async_copy(
                            jtab3.at[0], jblk.at[slot], sem).wait()
                        pltpu.make_async_copy(
                            ctab3.at[0], cblk.at[slot], sem).wait()
                    vjp = vjdx[pl.ds(p * half + (g - 1) * 16, 16)]
                    vcp = vcdx[pl.ds(p * half + (g - 1) * 16, 16)]
                    _extract(vjp, jblk, jrows, g - 1)
                    _extract(vcp, cblk, crows, g - 1)

                return 0

            lax.fori_loop(0, hg, group_body, 0)
            for l in range(16):
                slot = ((hg - 1) % 2) * 16 + l
                pltpu.make_async_copy(jtab3.at[0], jblk.at[slot], sem).wait()
                pltpu.make_async_copy(ctab3.at[0], cblk.at[slot], sem).wait()
            vjp = vjdx[pl.ds(p * half + (hg - 1) * 16, 16)]
            vcp = vcdx[pl.ds(p * half + (hg - 1) * 16, 16)]
            _extract(vjp, jblk, jrows, hg - 1)
            _extract(vcp, cblk, crows, hg - 1)

            pltpu.sync_copy(jrows, jout.at[pl.ds(base + p * half, half)])
            pltpu.sync_copy(crows, cout.at[pl.ds(base + p * half, half)])

    return pl.kernel(
        body,
        out_type=(
            jax.ShapeDtypeStruct((B, D), jnp.float32),
            jax.ShapeDtypeStruct((B, D), jnp.float32),
        ),
        mesh=mesh,
        scratch_types=[
            pltpu.VMEM((b_per_w,), jnp.int32),
            pltpu.VMEM((b_per_w,), jnp.int32),
            pltpu.VMEM((32, 16, D), dtype),
            pltpu.VMEM((32, 16, D), dtype),
            pltpu.VMEM((b_per_w // 4, D), jnp.float32),
            pltpu.VMEM((b_per_w // 4, D), jnp.float32),
            pltpu.SemaphoreType.DMA,
        ],
        compiler_params=pltpu.CompilerParams(needs_layout_passes=False),
    )


def _mlp_body(xj_ref, xc_ref, w1_ref, b1_ref, w2_ref, b2_ref, w3_ref, b3_ref,
              out_ref):
    d = xj_ref.shape[1]
    w1 = w1_ref[...]
    h = xj_ref[...] @ w1[:d, :] + xc_ref[...] @ w1[d:, :] + b1_ref[...]
    h = jnp.maximum(h, 0.0)
    h = jnp.maximum(h @ w2_ref[...] + b2_ref[...], 0.0)
    s = jax.nn.sigmoid(h @ w3_ref[...] + b3_ref[...])
    out_ref[...] = s[:, 0]


def kernel(job_ids, candidate_ids, job_table, cand_table, W1, b1, W2, b2, W3,
           b3):
    B = job_ids.shape[0]
    NJ, D = job_table.shape
    NCAND = cand_table.shape[0]
    H1 = W1.shape[1]
    H2 = W2.shape[1]

    jtab3 = job_table.astype(jnp.bfloat16).reshape(NJ // 16, 16, D)
    ctab3 = cand_table.astype(jnp.bfloat16).reshape(NCAND // 16, 16, D)

    gather = _make_gather(B, D, jnp.bfloat16)
    job_emb, cand_emb = gather(job_ids, candidate_ids, jtab3, ctab3)

    blk = 4096
    grid = (B // blk,)
    score = pl.pallas_call(
        _mlp_body,
        grid=grid,
        in_specs=[
            pl.BlockSpec((blk, D), lambda i: (i, 0)),
            pl.BlockSpec((blk, D), lambda i: (i, 0)),
            pl.BlockSpec((2 * D, H1), lambda i: (0, 0)),
            pl.BlockSpec((1, H1), lambda i: (0, 0)),
            pl.BlockSpec((H1, H2), lambda i: (0, 0)),
            pl.BlockSpec((1, H2), lambda i: (0, 0)),
            pl.BlockSpec((H2, 1), lambda i: (0, 0)),
            pl.BlockSpec((1, 1), lambda i: (0, 0)),
        ],
        out_specs=pl.BlockSpec((blk,), lambda i: (i,)),
        out_shape=jax.ShapeDtypeStruct((B,), jnp.float32),
    )(job_emb, cand_emb, W1, b1.reshape(1, H1), W2, b2.reshape(1, H2), W3,
      b3.reshape(1, 1))
    return score
